# baseline (speedup 1.0000x reference)
import functools
import jax
import jax.numpy as jnp
from jax import lax
from jax.experimental import pallas as pl
from jax.experimental.pallas import tpu as pltpu
from jax.experimental.pallas import tpu_sc as plsc

B = 4096
L = 200
D = 64
TOTAL = B * L

_info = plsc.get_sparse_core_info()
NC = _info.num_cores
NS = _info.num_subcores
NW = NC * NS
T_PAIRS = L // 2

_mesh = plsc.VectorSubcoreMesh(core_axis_name="c", subcore_axis_name="s")


@functools.partial(
    pl.kernel,
    mesh=_mesh,
    out_type=jax.ShapeDtypeStruct((L, 8, 32, 8, 128), jnp.float32),
    scratch_types=[
        pltpu.VMEM((128,), jnp.int32),
        pltpu.VMEM((128,), jnp.int32),
        pltpu.VMEM((128, D), jnp.float32),
        pltpu.VMEM((128, D), jnp.float32),
        pltpu.VMEM((8, 8, 128), jnp.float32),
        pltpu.VMEM((8, 8, 128), jnp.float32),
        pltpu.SemaphoreType.DMA,
        pltpu.SemaphoreType.DMA,
        pltpu.SemaphoreType.DMA,
        pltpu.SemaphoreType.DMA,
    ],
    compiler_params=pltpu.CompilerParams(use_tc_tiling_on_sc=False,
                                         needs_layout_passes=False),
)
def _gather_kernel(idx5_hbm, u_hbm, out_hbm,
                   idx_a, idx_b, g_a, g_b, o_a, o_b,
                   gsem_a, gsem_b, wsem_a, wsem_b):
    bc = lax.axis_index("s") * NC + lax.axis_index("c")

    iot = lax.iota(jnp.int32, 16)
    frv = [(iot + 16 * c) >> 3 for c in range(4)]
    sv = [(iot + 16 * c) & 7 for c in range(4)]

    def idx_at(l):
        return idx5_hbm.at[l // 8, bc, l % 8]

    def out_at(l):
        return out_hbm.at[l, :, bc]

    def shuffle(g_v, o_v):
        def blk(bb, carry):
            for u8 in range(8):
                bl = bb * 8 + u8
                blsplat = jnp.zeros((16,), jnp.int32) + bl
                for c in range(4):
                    vals = g_v[bl, pl.ds(16 * c, 16)]
                    plsc.store_scatter(o_v, [frv[c], sv[c], blsplat], vals)
            return carry
        lax.fori_loop(0, 16, blk, 0)

    pltpu.sync_copy(idx_at(0), idx_a)
    pltpu.async_copy(u_hbm.at[idx_a], g_a, gsem_a)

    def body(t, carry):
        l0 = 2 * t
        l1 = l0 + 1

        pltpu.sync_copy(idx_at(l1), idx_b)
        pltpu.async_copy(u_hbm.at[idx_b], g_b, gsem_b)

        pltpu.make_async_copy(u_hbm.at[idx_a], g_a, gsem_a).wait()

        @pl.when(t > 0)
        def _():
            pltpu.make_async_copy(o_a, out_at(l0), wsem_a).wait()

        shuffle(g_a, o_a)
        pltpu.async_copy(o_a, out_at(l0), wsem_a)

        @pl.when(t < T_PAIRS - 1)
        def _():
            pltpu.sync_copy(idx_at(l0 + 2), idx_a)
            pltpu.async_copy(u_hbm.at[idx_a], g_a, gsem_a)

        pltpu.make_async_copy(u_hbm.at[idx_b], g_b, gsem_b).wait()

        @pl.when(t > 0)
        def _():
            pltpu.make_async_copy(o_b, out_at(l1), wsem_b).wait()

        shuffle(g_b, o_b)
        pltpu.async_copy(o_b, out_at(l1), wsem_b)
        return carry

    lax.fori_loop(0, T_PAIRS, body, 0)
    pltpu.make_async_copy(o_a, out_at(L - 2), wsem_a).wait()
    pltpu.make_async_copy(o_b, out_at(L - 1), wsem_b).wait()


def _mask_body(idx_ref, out_ref):
    out_ref[...] = (idx_ref[...] > 0).astype(jnp.int32)


_mask = pl.pallas_call(
    _mask_body,
    out_shape=jax.ShapeDtypeStruct((6400, 128), jnp.int32),
    grid=(8,),
    in_specs=[pl.BlockSpec((800, 128), lambda i: (i, 0))],
    out_specs=pl.BlockSpec((800, 128), lambda i: (i, 0)),
)


@jax.jit
def kernel(word_idx, tune_pre_trained, table):
    wi = word_idx.astype(jnp.int32)
    idx5 = wi.T.reshape(25, 8, 32, 128).transpose(0, 2, 1, 3)
    ol = _gather_kernel(idx5, table)
    static_emb = ol.transpose(2, 4, 0, 1, 3).reshape(B, L, D)
    mask = _mask(wi.reshape(6400, 128))
    bottom_existence = mask.reshape(B, L, 1).astype(jnp.bool_)
    return (static_emb, bottom_existence)


# speedup vs baseline: 1.5326x; 1.5326x over previous
import functools
import jax
import jax.numpy as jnp
from jax import lax
from jax.experimental import pallas as pl
from jax.experimental.pallas import tpu as pltpu
from jax.experimental.pallas import tpu_sc as plsc

B = 4096
L = 200
D = 64
TOTAL = B * L

_info = plsc.get_sparse_core_info()
NC = _info.num_cores
NS = _info.num_subcores
NW = NC * NS
T_PAIRS = L // 2

_mesh = plsc.VectorSubcoreMesh(core_axis_name="c", subcore_axis_name="s")


@functools.partial(
    pl.kernel,
    mesh=_mesh,
    out_type=jax.ShapeDtypeStruct((L, 8, 32, 8, 128), jnp.float32),
    scratch_types=[
        pltpu.VMEM((128,), jnp.int32),
        pltpu.VMEM((128,), jnp.int32),
        pltpu.VMEM((128, D), jnp.float32),
        pltpu.VMEM((128, D), jnp.float32),
        pltpu.VMEM((8, 8, 129), jnp.float32),
        pltpu.VMEM((8, 8, 129), jnp.float32),
        pltpu.SemaphoreType.DMA,
        pltpu.SemaphoreType.DMA,
        pltpu.SemaphoreType.DMA,
        pltpu.SemaphoreType.DMA,
    ],
    compiler_params=pltpu.CompilerParams(use_tc_tiling_on_sc=False,
                                         needs_layout_passes=False),
)
def _gather_kernel(idx5_hbm, u_hbm, out_hbm,
                   idx_a, idx_b, g_a, g_b, o_a, o_b,
                   gsem_a, gsem_b, wsem_a, wsem_b):
    bc = lax.axis_index("s") * NC + lax.axis_index("c")

    iot = lax.iota(jnp.int32, 16)
    frv = [(iot + 16 * c) >> 3 for c in range(4)]
    sv = [(iot + 16 * c) & 7 for c in range(4)]

    def idx_at(l):
        return idx5_hbm.at[l // 8, bc, l % 8]

    def out_at(l):
        return out_hbm.at[l, :, bc]

    def shuffle(g_v, o_v):
        def blk(bb, carry):
            for u8 in range(8):
                bl = bb * 8 + u8
                blsplat = jnp.zeros((16,), jnp.int32) + bl
                for c in range(4):
                    vals = g_v[bl, pl.ds(16 * c, 16)]
                    plsc.store_scatter(o_v, [frv[c], sv[c], blsplat], vals)
            return carry
        lax.fori_loop(0, 16, blk, 0)

    pltpu.sync_copy(idx_at(0), idx_a)
    pltpu.async_copy(u_hbm.at[idx_a], g_a, gsem_a)

    def body(t, carry):
        l0 = 2 * t
        l1 = l0 + 1

        pltpu.sync_copy(idx_at(l1), idx_b)
        pltpu.async_copy(u_hbm.at[idx_b], g_b, gsem_b)

        pltpu.make_async_copy(u_hbm.at[idx_a], g_a, gsem_a).wait()

        @pl.when(t > 0)
        def _():
            pltpu.make_async_copy(o_a.at[:, :, pl.ds(0, 128)], out_at(l0), wsem_a).wait()

        shuffle(g_a, o_a)
        pltpu.async_copy(o_a.at[:, :, pl.ds(0, 128)], out_at(l0), wsem_a)

        @pl.when(t < T_PAIRS - 1)
        def _():
            pltpu.sync_copy(idx_at(l0 + 2), idx_a)
            pltpu.async_copy(u_hbm.at[idx_a], g_a, gsem_a)

        pltpu.make_async_copy(u_hbm.at[idx_b], g_b, gsem_b).wait()

        @pl.when(t > 0)
        def _():
            pltpu.make_async_copy(o_b.at[:, :, pl.ds(0, 128)], out_at(l1), wsem_b).wait()

        shuffle(g_b, o_b)
        pltpu.async_copy(o_b.at[:, :, pl.ds(0, 128)], out_at(l1), wsem_b)
        return carry

    lax.fori_loop(0, T_PAIRS, body, 0)
    pltpu.make_async_copy(o_a.at[:, :, pl.ds(0, 128)], out_at(L - 2), wsem_a).wait()
    pltpu.make_async_copy(o_b.at[:, :, pl.ds(0, 128)], out_at(L - 1), wsem_b).wait()


def _mask_body(idx_ref, out_ref):
    out_ref[...] = (idx_ref[...] > 0).astype(jnp.int32)


_mask = pl.pallas_call(
    _mask_body,
    out_shape=jax.ShapeDtypeStruct((6400, 128), jnp.int32),
    grid=(8,),
    in_specs=[pl.BlockSpec((800, 128), lambda i: (i, 0))],
    out_specs=pl.BlockSpec((800, 128), lambda i: (i, 0)),
)


@jax.jit
def kernel(word_idx, tune_pre_trained, table):
    wi = word_idx.astype(jnp.int32)
    idx5 = wi.T.reshape(25, 8, 32, 128).transpose(0, 2, 1, 3)
    ol = _gather_kernel(idx5, table)
    static_emb = ol.transpose(2, 4, 0, 1, 3).reshape(B, L, D)
    mask = _mask(wi.reshape(6400, 128))
    bottom_existence = mask.reshape(B, L, 1).astype(jnp.bool_)
    return (static_emb, bottom_existence)
